# baseline (speedup 1.0000x reference)
import functools

import numpy as np
import jax
import jax.numpy as jnp
from jax import lax
from jax.experimental import pallas as pl
from jax.experimental.pallas import tpu as pltpu
from jax.experimental.pallas import tpu_sc as plsc

_DIM = 64
_TOKENS = 16384
_LANES = 16

_info = plsc.get_sparse_core_info()
_NC = _info.num_cores
_NS = _info.num_subcores
_NW = _NC * _NS
_BPW = _TOKENS // _NW
_CH = 128
_NCH = _BPW // _CH


def _pe_table() -> np.ndarray:
    x = np.arange(_TOKENS, dtype=np.float32)[:, None]
    y = np.arange(_DIM, dtype=np.float32)[None, :]
    arg = x / np.float32(10000.0) ** (np.float32(2.0) * y / np.float32(_DIM))
    even = (np.arange(_TOKENS)[:, None] % 2) == 0
    return np.where(even, np.sin(arg), np.cos(arg)).astype(np.float32)


_PE = _pe_table()


def _sc_body(idx_hbm, pe_hbm, table_hbm, out_hbm, idx_v, pe_v, rows_v,
             gsem, psem):
    wid = lax.axis_index("s") * _NC + lax.axis_index("c")
    base = wid * _BPW
    pe_cp = pltpu.make_async_copy(pe_hbm.at[pl.ds(base, _BPW)], pe_v, psem)
    pe_cp.start()
    pltpu.sync_copy(idx_hbm.at[wid], idx_v)
    gathers = [
        pltpu.make_async_copy(
            table_hbm.at[idx_v.at[k]],
            rows_v.at[pl.ds(k * _CH, _CH)],
            gsem,
        )
        for k in range(_NCH)
    ]
    for cp in gathers:
        cp.start()
    for cp in gathers:
        cp.wait()
    pe_cp.wait()

    def add_rows(i, carry):
        for r in range(8):
            for c in range(_DIM // _LANES):
                sl = (i * 8 + r, pl.ds(c * _LANES, _LANES))
                rows_v[sl] = rows_v[sl] + pe_v[sl]
        return carry

    lax.fori_loop(0, _BPW // 8, add_rows, 0)
    pltpu.sync_copy(rows_v, out_hbm.at[pl.ds(base, _BPW)])


_emb = functools.partial(
    pl.kernel,
    out_type=jax.ShapeDtypeStruct((_TOKENS, _DIM), jnp.float32),
    mesh=plsc.VectorSubcoreMesh(core_axis_name="c", subcore_axis_name="s"),
    scratch_types=[
        pltpu.VMEM((_NCH, _CH), jnp.int32),
        pltpu.VMEM((_BPW, _DIM), jnp.float32),
        pltpu.VMEM((_BPW, _DIM), jnp.float32),
        pltpu.SemaphoreType.DMA,
        pltpu.SemaphoreType.DMA,
    ],
    compiler_params=pltpu.CompilerParams(use_tc_tiling_on_sc=False),
)(_sc_body)


def kernel(input, table):
    idx3 = input.astype(jnp.int32).reshape(_NW, _NCH, _CH)
    pe = jnp.asarray(_PE)
    return _emb(idx3, pe, table)


# speedup vs baseline: 1.5339x; 1.5339x over previous
import functools

import numpy as np
import jax
import jax.numpy as jnp
from jax import lax
from jax.experimental import pallas as pl
from jax.experimental.pallas import tpu as pltpu
from jax.experimental.pallas import tpu_sc as plsc

_DIM = 64
_TOKENS = 16384
_LANES = 16
_RPG = 8

_info = plsc.get_sparse_core_info()
_NC = _info.num_cores
_NS = _info.num_subcores
_NW = _NC * _NS
_BPW = _TOKENS // _NW
_CH = 64
_NCH = _BPW // _CH


def _pe_table() -> np.ndarray:
    x = np.arange(_TOKENS, dtype=np.float32)[:, None]
    y = np.arange(_DIM, dtype=np.float32)[None, :]
    arg = x / np.float32(10000.0) ** (np.float32(2.0) * y / np.float32(_DIM))
    even = (np.arange(_TOKENS)[:, None] % 2) == 0
    return np.where(even, np.sin(arg), np.cos(arg)).astype(np.float32)


_PE = _pe_table()


def _sc_body(idx_hbm, pe_hbm, table_hbm, out_hbm,
             idx_v, tiles_v, pe_v, ob_v, gsem, psem):
    wid = lax.axis_index("s") * _NC + lax.axis_index("c")
    base = wid * _BPW
    pltpu.sync_copy(idx_hbm.at[pl.ds(base, _BPW)], idx_v)

    for k in range(_NCH):
        cb = k * _CH
        pe_cp = pltpu.make_async_copy(
            pe_hbm.at[pl.ds(base + cb, _CH)], pe_v, psem)
        pe_cp.start()

        def fire(g, carry, _cb=cb):
            v = idx_v[pl.ds(_cb + g * _LANES, _LANES)]
            tids = lax.shift_right_logical(v, 3)
            for l in range(_LANES):
                tok = g * _LANES + l
                pltpu.make_async_copy(
                    table_hbm.at[pl.ds(tids[l] * _RPG, _RPG)],
                    tiles_v.at[tok],
                    gsem,
                ).start()
            return carry

        lax.fori_loop(0, _CH // _LANES, fire, 0)
        pltpu.make_async_copy(
            table_hbm.at[pl.ds(0, _CH * _RPG)], tiles_v, gsem).wait()
        pe_cp.wait()

        def extract(g, carry, _cb=cb):
            v = idx_v[pl.ds(_cb + g * _LANES, _LANES)]
            subs = lax.bitwise_and(v, 7)
            for l in range(_LANES):
                tok = g * _LANES + l
                sub = subs[l]
                for c in range(_DIM // _LANES):
                    csl = pl.ds(c * _LANES, _LANES)
                    ob_v[tok, csl] = tiles_v[tok, sub, csl] + pe_v[tok, csl]
            return carry

        lax.fori_loop(0, _CH // _LANES, extract, 0)
        pltpu.sync_copy(ob_v, out_hbm.at[pl.ds(base + cb, _CH)])


_emb = functools.partial(
    pl.kernel,
    out_type=jax.ShapeDtypeStruct((_TOKENS, _DIM), jnp.float32),
    mesh=plsc.VectorSubcoreMesh(core_axis_name="c", subcore_axis_name="s"),
    scratch_types=[
        pltpu.VMEM((_BPW,), jnp.int32),
        pltpu.VMEM((_CH, _RPG, _DIM), jnp.float32),
        pltpu.VMEM((_CH, _DIM), jnp.float32),
        pltpu.VMEM((_CH, _DIM), jnp.float32),
        pltpu.SemaphoreType.DMA,
        pltpu.SemaphoreType.DMA,
    ],
)(_sc_body)


def kernel(input, table):
    idx = input.astype(jnp.int32)
    pe = jnp.asarray(_PE)
    return _emb(idx, pe, table)
